# baseline (speedup 1.0000x reference)
import jax
import jax.numpy as jnp
from jax.experimental import pallas as pl
from jax.experimental.pallas import tpu as pltpu

T, B, C, K = 512, 16, 4, 64
CK = C * K
H = 2 * K


def _body(scal_ref, em_ref, ftag_ref, ftagp_ref, ts_ref, trans_ref,
          transT_ref, start_ref, end_ref, out_ref, xem_lo_ref, xem_hi_ref):
    ts = ts_ref[...]
    nwmax = scal_ref[0, 0]
    mid = scal_ref[0, 1]
    maxL = scal_ref[0, 2]

    def blk2(ref, c0):
        z = jnp.zeros((K, K), jnp.float32)
        top = jnp.concatenate([jnp.exp(ref[c0]), z], axis=1)
        bot = jnp.concatenate([z, jnp.exp(ref[c0 + 1])], axis=1)
        return jnp.concatenate([top, bot], axis=0).astype(jnp.bfloat16)

    E_lo, E_hi = blk2(trans_ref, 0), blk2(trans_ref, 2)
    ET_lo, ET_hi = blk2(transT_ref, 0), blk2(transT_ref, 2)
    xem_lo_ref[...] = jnp.exp(em_ref[:, :, :H])
    xem_hi_ref[...] = jnp.exp(em_ref[:, :, H:])
    transblk = jnp.concatenate(
        [jnp.concatenate(
            [trans_ref[c] if c == r else jnp.zeros((K, K), jnp.float32)
             for c in range(C)], axis=1) for r in range(C)],
        axis=0).astype(jnp.bfloat16)
    endv = end_ref[...].reshape(1, 1, CK)
    tsb = ts.reshape(1, B, 1)

    def init(sl):
        a0 = start_ref[:, sl] + em_ref[0, :, sl]
        m0 = jnp.max(a0, axis=1, keepdims=True)
        return jnp.exp(a0 - m0), m0

    ua_lo, Ma_lo = init(slice(0, H))
    ua_hi, Ma_hi = init(slice(H, CK))
    ub_lo = jnp.broadcast_to(jnp.exp(end_ref[:, :H]), (B, H))
    ub_hi = jnp.broadcast_to(jnp.exp(end_ref[:, H:]), (B, H))
    Mb_lo = jnp.zeros((B, 1), jnp.float32)
    Mb_hi = jnp.zeros((B, 1), jnp.float32)

    SB = 8

    def window(w, carry):
        ua_lo, ua_hi, ub_lo, ub_hi, Ma_lo, Ma_hi, Mb_lo, Mb_hi, acc = carry

        t8 = SB * w
        em_slab = em_ref[pl.ds(t8, SB)]
        ftag_slab = ftag_ref[pl.ds(t8, SB)]
        ftagp_slab = ftagp_ref[pl.ds(t8, SB)]
        trow = jax.lax.broadcasted_iota(jnp.int32, (SB, B, 1), 0) + t8
        maskT = (trow < tsb).astype(jnp.float32)
        endm = (trow == tsb - 1).astype(jnp.float32)
        lane = jax.lax.broadcasted_iota(jnp.int32, (SB, B, CK), 2)
        tsel = jnp.concatenate(
            [jnp.broadcast_to(ftag_slab[:, :, c:c + 1], (SB, B, K))
             for c in range(C)], axis=2)
        tselp = jnp.concatenate(
            [jnp.broadcast_to(ftagp_slab[:, :, c:c + 1], (SB, B, K))
             for c in range(C)], axis=2)
        ohp = (lane == tselp).astype(jnp.bfloat16)
        R = jnp.dot(ohp.reshape(SB * B, CK), transblk,
                    preferred_element_type=jnp.float32).reshape(SB, B, CK)
        gval = (em_slab + R) * maskT + endv * endm
        acc = acc + jnp.sum(jnp.where(lane == tsel, gval, 0.0), axis=0)

        for j in range(4):
            tf = 1 + 4 * w + j
            tfi = jnp.minimum(tf, T - 1)
            mf = jnp.logical_and(tf < ts, tf <= mid)
            va_lo = jnp.dot(ua_lo.astype(jnp.bfloat16), E_lo,
                            preferred_element_type=jnp.float32)
            va_hi = jnp.dot(ua_hi.astype(jnp.bfloat16), E_hi,
                            preferred_element_type=jnp.float32)
            tb = maxL - 2 - 4 * w - j
            tbi = jnp.clip(tb + 1, 0, T - 1)
            mb = jnp.logical_and(tb >= mid, tb < ts - 1)
            vb_lo = jnp.dot((ub_lo * xem_lo_ref[tbi]).astype(jnp.bfloat16),
                            ET_lo, preferred_element_type=jnp.float32)
            vb_hi = jnp.dot((ub_hi * xem_hi_ref[tbi]).astype(jnp.bfloat16),
                            ET_hi, preferred_element_type=jnp.float32)
            ua_lo = jnp.where(mf, va_lo * xem_lo_ref[tfi], ua_lo)
            ua_hi = jnp.where(mf, va_hi * xem_hi_ref[tfi], ua_hi)
            ub_lo = jnp.where(mb, vb_lo, ub_lo)
            ub_hi = jnp.where(mb, vb_hi, ub_hi)
        outs = []
        for u, M in ((ua_lo, Ma_lo), (ua_hi, Ma_hi), (ub_lo, Mb_lo),
                     (ub_hi, Mb_hi)):
            m = jnp.max(u, axis=1, keepdims=True)
            outs.append((u * (1.0 / m), M + jnp.log(m)))
        return (outs[0][0], outs[1][0], outs[2][0], outs[3][0],
                outs[0][1], outs[1][1], outs[2][1], outs[3][1], acc)

    lane0 = jax.lax.broadcasted_iota(jnp.int32, (B, CK), 1)
    tsel0 = jnp.concatenate(
        [jnp.broadcast_to(ftag_ref[0][:, c:c + 1], (B, K)) for c in range(C)],
        axis=1)
    acc0 = jnp.where(lane0 == tsel0,
                     jnp.broadcast_to(start_ref[...], (B, CK)), 0.0)

    carry = jax.lax.fori_loop(0, nwmax, window,
                              (ua_lo, ua_hi, ub_lo, ub_hi,
                               Ma_lo, Ma_hi, Mb_lo, Mb_hi, acc0))
    ua_lo, ua_hi, ub_lo, ub_hi, Ma_lo, Ma_hi, Mb_lo, Mb_hi, acc = carry
    q_lo = ua_lo * ub_lo
    q_hi = ua_hi * ub_hi
    den = jnp.concatenate(
        [jnp.log(jnp.sum(q_lo[:, :K], axis=1, keepdims=True)) + Ma_lo + Mb_lo,
         jnp.log(jnp.sum(q_lo[:, K:], axis=1, keepdims=True)) + Ma_lo + Mb_lo,
         jnp.log(jnp.sum(q_hi[:, :K], axis=1, keepdims=True)) + Ma_hi + Mb_hi,
         jnp.log(jnp.sum(q_hi[:, K:], axis=1, keepdims=True)) + Ma_hi + Mb_hi],
        axis=1)

    num = jnp.concatenate(
        [jnp.sum(acc[:, K * c:K * (c + 1)], axis=1, keepdims=True)
         for c in range(C)], axis=1)

    out_ref[...] = num - den


@jax.jit
def kernel(emissions, tags, token_sizes, transitions, start_transitions,
           end_transitions):
    f32 = jnp.float32
    em3 = emissions.reshape(T, B, CK).astype(f32)
    ftag = (tags.astype(jnp.int32) +
            (K * jnp.arange(C, dtype=jnp.int32))[None, None, :])
    tsB = token_sizes.astype(jnp.int32).reshape(B, 1)
    startblk = start_transitions.reshape(1, CK).astype(f32)
    endblk = end_transitions.reshape(1, CK).astype(f32)
    ftagp = jnp.concatenate(
        [jnp.full((1, B, C), -1, jnp.int32), ftag[:-1]], axis=0)
    maxL = jnp.max(token_sizes.astype(jnp.int32))
    nfwin = ((maxL - 1) // 2 + 3) // 4
    mid = 4 * nfwin
    nbwin = (jnp.maximum(maxL - 1 - mid, 0) + 3) // 4
    nwmax = jnp.maximum(jnp.maximum(nfwin, nbwin), (maxL + 7) // 8)
    scal = jnp.stack([nwmax, mid, maxL]).reshape(1, 3)

    return pl.pallas_call(
        _body,
        in_specs=[
            pl.BlockSpec(memory_space=pltpu.SMEM),
            pl.BlockSpec((T, B, CK), lambda: (0, 0, 0)),
            pl.BlockSpec((T, B, C), lambda: (0, 0, 0)),
            pl.BlockSpec((T, B, C), lambda: (0, 0, 0)),
            pl.BlockSpec((B, 1), lambda: (0, 0)),
            pl.BlockSpec((C, K, K), lambda: (0, 0, 0)),
            pl.BlockSpec((C, K, K), lambda: (0, 0, 0)),
            pl.BlockSpec((1, CK), lambda: (0, 0)),
            pl.BlockSpec((1, CK), lambda: (0, 0)),
        ],
        out_specs=pl.BlockSpec((B, C), lambda: (0, 0)),
        out_shape=jax.ShapeDtypeStruct((B, C), f32),
        scratch_shapes=[pltpu.VMEM((T, B, H), f32) for _ in range(2)],
    )(scal, em3, ftag, ftagp, tsB, transitions.astype(f32),
      jnp.transpose(transitions, (0, 2, 1)).astype(f32), startblk, endblk)


# speedup vs baseline: 1.1005x; 1.1005x over previous
import jax
import jax.numpy as jnp
from jax.experimental import pallas as pl
from jax.experimental.pallas import tpu as pltpu

T, B, C, K = 512, 16, 4, 64
CK = C * K
H = 2 * K


def _body(scal_ref, em_ref, ftag_ref, ts_ref, trans_ref, transT_ref,
          start_ref, end_ref, out_ref, xem_lo_ref, xem_hi_ref):
    ts = ts_ref[...]
    nwmax = scal_ref[0, 0]
    mid = scal_ref[0, 1]
    maxL = scal_ref[0, 2]

    def blk2(ref, c0):
        z = jnp.zeros((K, K), jnp.float32)
        top = jnp.concatenate([jnp.exp(ref[c0]), z], axis=1)
        bot = jnp.concatenate([z, jnp.exp(ref[c0 + 1])], axis=1)
        return jnp.concatenate([top, bot], axis=0).astype(jnp.bfloat16)

    E_lo, E_hi = blk2(trans_ref, 0), blk2(trans_ref, 2)
    ET_lo, ET_hi = blk2(transT_ref, 0), blk2(transT_ref, 2)
    xem_lo_ref[...] = jnp.exp(em_ref[:, :, :H])
    xem_hi_ref[...] = jnp.exp(em_ref[:, :, H:])

    def init(sl):
        a0 = start_ref[:, sl] + em_ref[0, :, sl]
        m0 = jnp.max(a0, axis=1, keepdims=True)
        return jnp.exp(a0 - m0), m0

    ua_lo, Ma_lo = init(slice(0, H))
    ua_hi, Ma_hi = init(slice(H, CK))
    ub_lo = jnp.broadcast_to(jnp.exp(end_ref[:, :H]), (B, H))
    ub_hi = jnp.broadcast_to(jnp.exp(end_ref[:, H:]), (B, H))
    Mb_lo = jnp.zeros((B, 1), jnp.float32)
    Mb_hi = jnp.zeros((B, 1), jnp.float32)

    def window(w, carry):
        ua_lo, ua_hi, ub_lo, ub_hi, Ma_lo, Ma_hi, Mb_lo, Mb_hi = carry
        for j in range(4):
            tf = 1 + 4 * w + j
            tfi = jnp.minimum(tf, T - 1)
            mf = jnp.logical_and(tf < ts, tf <= mid)
            va_lo = jnp.dot(ua_lo.astype(jnp.bfloat16), E_lo,
                            preferred_element_type=jnp.float32)
            va_hi = jnp.dot(ua_hi.astype(jnp.bfloat16), E_hi,
                            preferred_element_type=jnp.float32)
            tb = maxL - 2 - 4 * w - j
            tbi = jnp.clip(tb + 1, 0, T - 1)
            mb = jnp.logical_and(tb >= mid, tb < ts - 1)
            vb_lo = jnp.dot((ub_lo * xem_lo_ref[tbi]).astype(jnp.bfloat16),
                            ET_lo, preferred_element_type=jnp.float32)
            vb_hi = jnp.dot((ub_hi * xem_hi_ref[tbi]).astype(jnp.bfloat16),
                            ET_hi, preferred_element_type=jnp.float32)
            ua_lo = jnp.where(mf, va_lo * xem_lo_ref[tfi], ua_lo)
            ua_hi = jnp.where(mf, va_hi * xem_hi_ref[tfi], ua_hi)
            ub_lo = jnp.where(mb, vb_lo, ub_lo)
            ub_hi = jnp.where(mb, vb_hi, ub_hi)
        outs = []
        for u, M in ((ua_lo, Ma_lo), (ua_hi, Ma_hi), (ub_lo, Mb_lo),
                     (ub_hi, Mb_hi)):
            m = jnp.max(u, axis=1, keepdims=True)
            outs.append((u * (1.0 / m), M + jnp.log(m)))
        return (outs[0][0], outs[1][0], outs[2][0], outs[3][0],
                outs[0][1], outs[1][1], outs[2][1], outs[3][1])

    carry = jax.lax.fori_loop(0, nwmax, window,
                              (ua_lo, ua_hi, ub_lo, ub_hi,
                               Ma_lo, Ma_hi, Mb_lo, Mb_hi))
    ua_lo, ua_hi, ub_lo, ub_hi, Ma_lo, Ma_hi, Mb_lo, Mb_hi = carry
    q_lo = ua_lo * ub_lo
    q_hi = ua_hi * ub_hi
    den = jnp.concatenate(
        [jnp.log(jnp.sum(q_lo[:, :K], axis=1, keepdims=True)) + Ma_lo + Mb_lo,
         jnp.log(jnp.sum(q_lo[:, K:], axis=1, keepdims=True)) + Ma_lo + Mb_lo,
         jnp.log(jnp.sum(q_hi[:, :K], axis=1, keepdims=True)) + Ma_hi + Mb_hi,
         jnp.log(jnp.sum(q_hi[:, K:], axis=1, keepdims=True)) + Ma_hi + Mb_hi],
        axis=1)

    trow = jax.lax.broadcasted_iota(jnp.int32, (T, B, 1), 0)
    tsb = ts.reshape(1, B, 1)
    maskT = (trow < tsb).astype(jnp.float32)
    endm = (trow == tsb - 1).astype(jnp.float32)

    lane = jax.lax.broadcasted_iota(jnp.int32, (T, B, CK), 2)
    tsel = jnp.concatenate(
        [jnp.broadcast_to(ftag_ref[:, :, c:c + 1], (T, B, K))
         for c in range(C)], axis=2)
    oh = (lane == tsel).astype(jnp.bfloat16)
    ohp = jnp.concatenate(
        [jnp.zeros((1, B, CK), jnp.bfloat16), oh[:-1]], axis=0)
    transblk = jnp.concatenate(
        [jnp.concatenate(
            [trans_ref[c] if c == r else jnp.zeros((K, K), jnp.float32)
             for c in range(C)], axis=1) for r in range(C)],
        axis=0).astype(jnp.bfloat16)
    R = jnp.dot(ohp.reshape(T * B, CK), transblk,
                preferred_element_type=jnp.float32).reshape(T, B, CK)
    gval = ((em_ref[...] + R) * maskT +
            end_ref[...].reshape(1, 1, CK) * endm)
    tot = jnp.sum(jnp.where(lane == tsel, gval, 0.0), axis=0)
    tot = tot + jnp.where(lane[0] == tsel[0],
                          jnp.broadcast_to(start_ref[...], (B, CK)), 0.0)
    num = jnp.concatenate(
        [jnp.sum(tot[:, K * c:K * (c + 1)], axis=1, keepdims=True)
         for c in range(C)], axis=1)

    out_ref[...] = num - den


@jax.jit
def kernel(emissions, tags, token_sizes, transitions, start_transitions,
           end_transitions):
    f32 = jnp.float32
    em3 = emissions.reshape(T, B, CK).astype(f32)
    ftag = (tags.astype(jnp.int32) +
            (K * jnp.arange(C, dtype=jnp.int32))[None, None, :])
    tsB = token_sizes.astype(jnp.int32).reshape(B, 1)
    startblk = start_transitions.reshape(1, CK).astype(f32)
    endblk = end_transitions.reshape(1, CK).astype(f32)
    maxL = jnp.max(token_sizes.astype(jnp.int32))
    nfwin = ((maxL - 1) // 2 + 3) // 4
    mid = 4 * nfwin
    nbwin = (jnp.maximum(maxL - 1 - mid, 0) + 3) // 4
    nwmax = jnp.maximum(nfwin, nbwin)
    scal = jnp.stack([nwmax, mid, maxL]).reshape(1, 3)

    return pl.pallas_call(
        _body,
        in_specs=[
            pl.BlockSpec(memory_space=pltpu.SMEM),
            pl.BlockSpec((T, B, CK), lambda: (0, 0, 0)),
            pl.BlockSpec((T, B, C), lambda: (0, 0, 0)),
            pl.BlockSpec((B, 1), lambda: (0, 0)),
            pl.BlockSpec((C, K, K), lambda: (0, 0, 0)),
            pl.BlockSpec((C, K, K), lambda: (0, 0, 0)),
            pl.BlockSpec((1, CK), lambda: (0, 0)),
            pl.BlockSpec((1, CK), lambda: (0, 0)),
        ],
        out_specs=pl.BlockSpec((B, C), lambda: (0, 0)),
        out_shape=jax.ShapeDtypeStruct((B, C), f32),
        scratch_shapes=[pltpu.VMEM((T, B, H), f32) for _ in range(2)],
    )(scal, em3, ftag, tsB, transitions.astype(f32),
      jnp.transpose(transitions, (0, 2, 1)).astype(f32), startblk, endblk)
